# baseline (device time: 8591 ns/iter reference)
import jax
import jax.numpy as jnp
from jax import lax
from jax.experimental import pallas as pl
from jax.experimental.pallas import tpu as pltpu

N_CHUNKS = 2


def kernel(x, gamma, beta):
    m, n = x.shape
    mc = m // N_CHUNKS
    n_global = 2 * n
    eps = 1e-5

    def body(x_hbm, g_hbm, b_hbm, out_hbm,
             xbuf, gbuf, obuf, stats_send, stats_recv,
             load_sems, gb_sem, store_sems, send_sems, recv_sems):
        my_x = lax.axis_index("x")
        my_y = lax.axis_index("y")
        peer = (my_x, 1 - my_y)

        barrier_sem = pltpu.get_barrier_semaphore()
        pl.semaphore_signal(
            barrier_sem, inc=1,
            device_id=peer, device_id_type=pl.DeviceIdType.MESH,
        )

        loads = []
        for c in range(N_CHUNKS):
            ld = pltpu.make_async_copy(
                x_hbm.at[pl.ds(c * mc, mc)], xbuf.at[c], load_sems.at[c]
            )
            ld.start()
            loads.append(ld)
        ld_g = pltpu.make_async_copy(g_hbm, gbuf.at[0], gb_sem.at[0])
        ld_g.start()
        ld_b = pltpu.make_async_copy(b_hbm, gbuf.at[1], gb_sem.at[1])
        ld_b.start()

        rdmas = []
        xs = []
        for c in range(N_CHUNKS):
            loads[c].wait()
            xv = xbuf[c]
            xs.append(xv)
            stats_send[c, :, 0:1] = jnp.sum(xv, axis=1, keepdims=True)
            stats_send[c, :, 1:2] = jnp.sum(xv * xv, axis=1, keepdims=True)
            if c == 0:
                pl.semaphore_wait(barrier_sem, 1)
            rdma = pltpu.make_async_remote_copy(
                src_ref=stats_send.at[c],
                dst_ref=stats_recv.at[c],
                send_sem=send_sems.at[c],
                recv_sem=recv_sems.at[c],
                device_id=peer,
                device_id_type=pl.DeviceIdType.MESH,
            )
            rdma.start()
            rdmas.append(rdma)

        ld_g.wait()
        ld_b.wait()
        g = gbuf[0]
        b = gbuf[1]

        stores = []
        for c in range(N_CHUNKS):
            xg = xs[c] * g
            rdmas[c].wait_recv()
            total = stats_send[c, :, :] + stats_recv[c, :, :]
            mean = total[:, 0:1] / n_global
            var = total[:, 1:2] / n_global - mean * mean
            inv = lax.rsqrt(var + eps)
            obuf[c, :, :] = (xg - mean * g) * inv + b
            st = pltpu.make_async_copy(
                obuf.at[c], out_hbm.at[pl.ds(c * mc, mc)], store_sems.at[c]
            )
            st.start()
            stores.append(st)

        for c in range(N_CHUNKS):
            stores[c].wait()
            rdmas[c].wait_send()

    hbm = pltpu.MemorySpace.HBM
    x = pltpu.with_memory_space_constraint(x, hbm)
    gamma = pltpu.with_memory_space_constraint(gamma.reshape(1, n), hbm)
    beta = pltpu.with_memory_space_constraint(beta.reshape(1, n), hbm)

    return pl.pallas_call(
        body,
        out_shape=jax.ShapeDtypeStruct((m, n), jnp.float32),
        in_specs=[
            pl.BlockSpec(memory_space=pl.ANY),
            pl.BlockSpec(memory_space=pl.ANY),
            pl.BlockSpec(memory_space=pl.ANY),
        ],
        out_specs=pl.BlockSpec(memory_space=pl.ANY),
        scratch_shapes=[
            pltpu.VMEM((N_CHUNKS, mc, n), jnp.float32),
            pltpu.VMEM((2, 1, n), jnp.float32),
            pltpu.VMEM((N_CHUNKS, mc, n), jnp.float32),
            pltpu.VMEM((N_CHUNKS, mc, 2), jnp.float32),
            pltpu.VMEM((N_CHUNKS, mc, 2), jnp.float32),
            pltpu.SemaphoreType.DMA((N_CHUNKS,)),
            pltpu.SemaphoreType.DMA((2,)),
            pltpu.SemaphoreType.DMA((N_CHUNKS,)),
            pltpu.SemaphoreType.DMA((N_CHUNKS,)),
            pltpu.SemaphoreType.DMA((N_CHUNKS,)),
        ],
        compiler_params=pltpu.CompilerParams(collective_id=0),
    )(x, gamma, beta)


# device time: 8485 ns/iter; 1.0125x vs baseline; 1.0125x over previous
import jax
import jax.numpy as jnp
from jax import lax
from jax.experimental import pallas as pl
from jax.experimental.pallas import tpu as pltpu

N_CHUNKS = 4


def kernel(x, gamma, beta):
    m, n = x.shape
    mc = m // N_CHUNKS
    n_global = 2 * n
    eps = 1e-5

    def body(x_hbm, g_hbm, b_hbm, out_ref,
             xbuf, gbuf, stats_send, stats_recv,
             load_sems, gb_sem, send_sems, recv_sems):
        my_x = lax.axis_index("x")
        my_y = lax.axis_index("y")
        peer = (my_x, 1 - my_y)

        barrier_sem = pltpu.get_barrier_semaphore()
        pl.semaphore_signal(
            barrier_sem, inc=1,
            device_id=peer, device_id_type=pl.DeviceIdType.MESH,
        )

        loads = []
        for c in range(N_CHUNKS):
            ld = pltpu.make_async_copy(
                x_hbm.at[pl.ds(c * mc, mc)], xbuf.at[c], load_sems.at[c]
            )
            ld.start()
            loads.append(ld)
        ld_g = pltpu.make_async_copy(g_hbm, gbuf.at[0], gb_sem.at[0])
        ld_g.start()
        ld_b = pltpu.make_async_copy(b_hbm, gbuf.at[1], gb_sem.at[1])
        ld_b.start()

        rdmas = []
        xs = []
        for c in range(N_CHUNKS):
            loads[c].wait()
            xv = xbuf[c]
            xs.append(xv)
            stats_send[c, :, 0:1] = jnp.sum(xv, axis=1, keepdims=True)
            stats_send[c, :, 1:2] = jnp.sum(xv * xv, axis=1, keepdims=True)
            if c == 0:
                pl.semaphore_wait(barrier_sem, 1)
            rdma = pltpu.make_async_remote_copy(
                src_ref=stats_send.at[c],
                dst_ref=stats_recv.at[c],
                send_sem=send_sems.at[c],
                recv_sem=recv_sems.at[c],
                device_id=peer,
                device_id_type=pl.DeviceIdType.MESH,
            )
            rdma.start()
            rdmas.append(rdma)

        ld_g.wait()
        ld_b.wait()
        g = gbuf[0]
        b = gbuf[1]

        for c in range(N_CHUNKS):
            xg = xs[c] * g
            rdmas[c].wait_recv()
            total = stats_send[c, :, :] + stats_recv[c, :, :]
            mean = total[:, 0:1] / n_global
            var = total[:, 1:2] / n_global - mean * mean
            inv = lax.rsqrt(var + eps)
            out_ref[pl.ds(c * mc, mc), :] = (xg - mean * g) * inv + b

        for c in range(N_CHUNKS):
            rdmas[c].wait_send()

    hbm = pltpu.MemorySpace.HBM
    x = pltpu.with_memory_space_constraint(x, hbm)
    gamma = pltpu.with_memory_space_constraint(gamma.reshape(1, n), hbm)
    beta = pltpu.with_memory_space_constraint(beta.reshape(1, n), hbm)

    return pl.pallas_call(
        body,
        out_shape=jax.ShapeDtypeStruct((m, n), jnp.float32),
        in_specs=[
            pl.BlockSpec(memory_space=pl.ANY),
            pl.BlockSpec(memory_space=pl.ANY),
            pl.BlockSpec(memory_space=pl.ANY),
        ],
        out_specs=pl.BlockSpec(memory_space=pltpu.VMEM),
        scratch_shapes=[
            pltpu.VMEM((N_CHUNKS, mc, n), jnp.float32),
            pltpu.VMEM((2, 1, n), jnp.float32),
            pltpu.VMEM((N_CHUNKS, mc, 2), jnp.float32),
            pltpu.VMEM((N_CHUNKS, mc, 2), jnp.float32),
            pltpu.SemaphoreType.DMA((N_CHUNKS,)),
            pltpu.SemaphoreType.DMA((2,)),
            pltpu.SemaphoreType.DMA((N_CHUNKS,)),
            pltpu.SemaphoreType.DMA((N_CHUNKS,)),
        ],
        compiler_params=pltpu.CompilerParams(collective_id=0),
    )(x, gamma, beta)
